# Initial kernel scaffold; baseline (speedup 1.0000x reference)
import jax
import jax.numpy as jnp
from jax.experimental import pallas as pl


def kernel(point_cloud, normals, colors, grid_points, params):
    raise NotImplementedError("write your pallas kernel here")


# speedup vs baseline: 2.1201x; 2.1201x over previous
import jax
import jax.numpy as jnp
from jax.experimental import pallas as pl

B = 8
M = 64
G = 64
F = 192


def _qc_kernel(x_ref, gp_ref,
               w1_ref, b1_ref, w2_ref, b2_ref, w3_ref, b3_ref,
               dnw1_ref, dnb1_ref, dnw2_ref, dnb2_ref,
               clw1_ref, clb1_ref, clw2_ref, clb2_ref,
               svw1_ref, svb1_ref, svw2_ref, svb2_ref,
               probs_ref, sev_ref, proc_ref, gft_ref):
    x = x_ref[...]
    gp = gp_ref[...]

    h = jnp.maximum(x @ w1_ref[...] + b1_ref[...], 0.0)
    h = jnp.maximum(h @ w2_ref[...] + b2_ref[...], 0.0)
    comb = h @ w3_ref[...] + b3_ref[...]

    d = ((x[:, 0:1] - gp[0:1, :]) ** 2
         + (x[:, 1:2] - gp[1:2, :]) ** 2
         + (x[:, 2:3] - gp[2:3, :]) ** 2)
    minv = jnp.min(d, axis=1, keepdims=True)
    gio = jax.lax.broadcasted_iota(jnp.int32, (B * M, G), 1)
    idxc = jnp.min(jnp.where(d == minv, gio, G), axis=1, keepdims=True)

    gio64 = jax.lax.broadcasted_iota(jnp.int32, (M, G), 1)
    rio64 = jax.lax.broadcasted_iota(jnp.int32, (M, G), 0)

    dnw1 = dnw1_ref[...]
    dnb1 = dnb1_ref[...]
    dnw2 = dnw2_ref[...]
    dnb2 = dnb2_ref[...]
    clw1 = clw1_ref[...]
    clb1 = clb1_ref[...]
    clw2 = clw2_ref[...]
    clb2 = clb2_ref[...]
    svw1 = svw1_ref[...]
    svb1 = svb1_ref[...]
    svw2 = svw2_ref[...]
    svb2 = svb2_ref[...]

    for b in range(B):
        idx_b = idxc[b * M:(b + 1) * M]
        onehot = idx_b == gio64
        val = jnp.where(onehot, rio64 + 1, 0)
        wins = jnp.max(val, axis=0, keepdims=True)
        selT = ((val == wins) & (wins > 0)).astype(jnp.float32)
        comb_b = comb[b * M:(b + 1) * M]
        gf = jax.lax.dot_general(
            selT, comb_b, (((0,), (0,)), ((), ())),
            preferred_element_type=jnp.float32)

        hd = jnp.maximum(gf @ dnw1 + dnb1, 0.0)
        defect = hd @ dnw2 + dnb2

        gft_ref[b, :, :] = gf.T
        proc_ref[b, :, :] = defect.T

        hc = jnp.maximum(defect @ clw1 + clb1, 0.0)
        logits = hc @ clw2 + clb2
        probs_ref[b, :, :] = jax.nn.softmax(logits, axis=-1)

        hs = jnp.maximum(defect @ svw1 + svb1, 0.0)
        sev_ref[b, :, :] = jax.nn.sigmoid(hs @ svw2 + svb2)


def _block_diag3(a, b, c):
    ra, ca = a.shape
    rb, cb = b.shape
    rc, cc = c.shape
    out = jnp.zeros((ra + rb + rc, ca + cb + cc), a.dtype)
    out = out.at[:ra, :ca].set(a)
    out = out.at[ra:ra + rb, ca:ca + cb].set(b)
    out = out.at[ra + rb:, ca + cb:].set(c)
    return out


def kernel(point_cloud, normals, colors, grid_points, params):
    x = jnp.concatenate(
        [point_cloud[:, :M], normals[:, :M], colors[:, :M]], axis=-1
    ).reshape(B * M, 9)
    gp_t = grid_points.T

    w1 = _block_diag3(params["pt_W1"], params["nm_W1"], params["tx_W1"])
    w2 = _block_diag3(params["pt_W2"], params["nm_W2"], params["tx_W2"])
    w3 = _block_diag3(params["pt_W3"], params["nm_W3"], params["tx_W3"])
    b1 = jnp.concatenate(
        [params["pt_b1"], params["nm_b1"], params["tx_b1"]]).reshape(1, -1)
    b2 = jnp.concatenate(
        [params["pt_b2"], params["nm_b2"], params["tx_b2"]]).reshape(1, -1)
    b3 = jnp.concatenate(
        [params["pt_b3"], params["nm_b3"], params["tx_b3"]]).reshape(1, -1)

    row = lambda v: v.reshape(1, -1)

    out_shapes = (
        jax.ShapeDtypeStruct((B, G, 5), jnp.float32),
        jax.ShapeDtypeStruct((B, G, 1), jnp.float32),
        jax.ShapeDtypeStruct((B, 64, G), jnp.float32),
        jax.ShapeDtypeStruct((B, F, G), jnp.float32),
    )

    probs, sev, proc, gft = pl.pallas_call(
        _qc_kernel,
        out_shape=out_shapes,
    )(x, gp_t,
      w1, b1, w2, b2, w3, b3,
      params["dn_W1"], row(params["dn_b1"]),
      params["dn_W2"], row(params["dn_b2"]),
      params["cl_W1"], row(params["cl_b1"]),
      params["cl_W2"], row(params["cl_b2"]),
      params["sv_W1"], row(params["sv_b1"]),
      params["sv_W2"], row(params["sv_b2"]))

    return probs, sev[..., 0], proc, gft
